# baseline (speedup 1.0000x reference)
import jax
import jax.numpy as jnp
from jax import lax
from jax.experimental import pallas as pl
from jax.experimental.pallas import tpu as pltpu
from jax.experimental.pallas import tpu_sc as plsc

B = 16384
D = 32
NC = 2
NS = 16
NW = NC * NS
BPW = B // NW
CHUNK = 128
NCHUNK = BPW // CHUNK
L = 16


def _sc_body(uid_hbm, iid_hbm, ue_hbm, ie_hbm, ub_hbm, ib_hbm, out_hbm,
             uid_v, iid_v, ue_v, ie_v, ub_v, ib_v, out_v, sem):
  wid = lax.axis_index("s") * NC + lax.axis_index("c")
  base = pl.multiple_of(wid * BPW, BPW)

  pltpu.sync_copy(uid_hbm.at[pl.ds(base, BPW)], uid_v)
  pltpu.sync_copy(iid_hbm.at[pl.ds(base, BPW)], iid_v)

  for j in range(NCHUNK):
    pltpu.async_copy(
        ub_hbm.at[uid_v.at[pl.ds(j * CHUNK, CHUNK)]],
        ub_v.at[pl.ds(j * CHUNK, CHUNK)], sem)
    pltpu.async_copy(
        ib_hbm.at[iid_v.at[pl.ds(j * CHUNK, CHUNK)]],
        ib_v.at[pl.ds(j * CHUNK, CHUNK)], sem)

  def fire(d, carry):
    for j in range(NCHUNK):
      idx_u = uid_v.at[pl.ds(j * CHUNK, CHUNK)]
      idx_i = iid_v.at[pl.ds(j * CHUNK, CHUNK)]
      pltpu.async_copy(ue_hbm.at[d].at[idx_u],
                       ue_v.at[d, pl.ds(j * CHUNK, CHUNK)], sem)
      pltpu.async_copy(ie_hbm.at[d].at[idx_i],
                       ie_v.at[d, pl.ds(j * CHUNK, CHUNK)], sem)
    return carry

  lax.fori_loop(0, D, fire, 0)

  pltpu.make_async_copy(ue_hbm.at[0], ue_v, sem).wait()
  pltpu.make_async_copy(ie_hbm.at[0], ie_v, sem).wait()
  pltpu.make_async_copy(ub_hbm.at[pl.ds(0, BPW)], ub_v, sem).wait()
  pltpu.make_async_copy(ib_hbm.at[pl.ds(0, BPW)], ib_v, sem).wait()

  def group(k, carry):
    s = pl.multiple_of(k * L, L)
    acc = ub_v[pl.ds(s, L)] + ib_v[pl.ds(s, L)]
    for d in range(D):
      acc = acc + ue_v[d, pl.ds(s, L)] * ie_v[d, pl.ds(s, L)]
    out_v[pl.ds(s, L)] = acc
    return carry

  lax.fori_loop(0, BPW // L, group, 0, unroll=2)

  pltpu.sync_copy(out_v, out_hbm.at[pl.ds(base, BPW)])


@jax.jit
def _run(user_ids, item_ids, user_emb_t, item_emb_t, user_bias, item_bias):
  mesh = plsc.VectorSubcoreMesh(
      core_axis_name="c", subcore_axis_name="s",
      num_cores=NC, num_subcores=NS)
  f = pl.kernel(
      _sc_body,
      out_type=jax.ShapeDtypeStruct((B,), jnp.float32),
      mesh=mesh,
      scratch_types=[
          pltpu.VMEM((BPW,), jnp.int32),
          pltpu.VMEM((BPW,), jnp.int32),
          pltpu.VMEM((D, BPW), jnp.float32),
          pltpu.VMEM((D, BPW), jnp.float32),
          pltpu.VMEM((BPW,), jnp.float32),
          pltpu.VMEM((BPW,), jnp.float32),
          pltpu.VMEM((BPW,), jnp.float32),
          pltpu.SemaphoreType.DMA,
      ],
      compiler_params=pltpu.CompilerParams(
          needs_layout_passes=False, use_tc_tiling_on_sc=False),
  )
  return f(user_ids, item_ids, user_emb_t, item_emb_t, user_bias, item_bias)


def kernel(user_ids, item_ids, user_emb, item_emb, user_bias, item_bias):
  return _run(user_ids.astype(jnp.int32), item_ids.astype(jnp.int32),
              user_emb.T, item_emb.T,
              user_bias.reshape(-1), item_bias.reshape(-1))


# speedup vs baseline: 15.1886x; 15.1886x over previous
import jax
import jax.numpy as jnp
from jax import lax
from jax.experimental import pallas as pl
from jax.experimental.pallas import tpu as pltpu
from jax.experimental.pallas import tpu_sc as plsc

B = 16384
D = 32
NC, NS = 2, 16
NW = NC * NS
BPW = B // NW
CHUNK = 128
NCHUNK = BPW // CHUNK
L = 16
NG = BPW // L
W = 128
HD = 16


def _sc_body(uid_hbm, iid_hbm, ue_hbm, ie_hbm, ub_hbm, ib_hbm, out_hbm,
             uid_v, iid_v, win_u, win_i, ptmp, ub_v, ib_v, out_v, sem, bsem):
  wid = lax.axis_index("s") * NC + lax.axis_index("c")
  base = pl.multiple_of(wid * BPW, BPW)

  pltpu.sync_copy(uid_hbm.at[pl.ds(base, BPW)], uid_v)
  pltpu.sync_copy(iid_hbm.at[pl.ds(base, BPW)], iid_v)

  for j in range(NCHUNK):
    pltpu.async_copy(ub_hbm.at[uid_v.at[pl.ds(j * CHUNK, CHUNK)]],
                     ub_v.at[pl.ds(j * CHUNK, CHUNK)], bsem)
    pltpu.async_copy(ib_hbm.at[iid_v.at[pl.ds(j * CHUNK, CHUNK)]],
                     ib_v.at[pl.ds(j * CHUNK, CHUNK)], bsem)
  pltpu.make_async_copy(ub_hbm.at[pl.ds(0, BPW)], ub_v, bsem).wait()
  pltpu.make_async_copy(ib_hbm.at[pl.ds(0, BPW)], ib_v, bsem).wait()

  lanes = lax.iota(jnp.int32, L)

  def fire(h, u16, i16):
    ublk = u16 & -128
    iblk = i16 & -128
    dh = h * HD
    for e in range(L):
      ou = pl.multiple_of(ublk[e], W)
      oi = pl.multiple_of(iblk[e], W)
      pltpu.async_copy(ue_hbm.at[pl.ds(dh, HD), pl.ds(ou, W)],
                       win_u.at[pl.ds(e * HD, HD)], sem)
      pltpu.async_copy(ie_hbm.at[pl.ds(dh, HD), pl.ds(oi, W)],
                       win_i.at[pl.ds(e * HD, HD)], sem)

  def run_pass(h):
    fire(h, uid_v[pl.ds(0, L)], iid_v[pl.ds(0, L)])

    def group(g, carry):
      s = pl.multiple_of(g * L, L)
      u16 = uid_v[pl.ds(s, L)]
      i16 = iid_v[pl.ds(s, L)]
      uoff = u16 & 127
      ioff = i16 & 127
      pltpu.make_async_copy(ue_hbm.at[pl.ds(0, HD), pl.ds(0, W)],
                            win_u, sem).wait()
      pltpu.make_async_copy(ie_hbm.at[pl.ds(0, HD), pl.ds(0, W)],
                            win_i, sem).wait()
      for e in range(L):
        ru = e * HD + lanes
        uc = plsc.load_gather(win_u, [ru, jnp.full((L,), uoff[e], jnp.int32)])
        ic = plsc.load_gather(win_i, [ru, jnp.full((L,), ioff[e], jnp.int32)])
        ptmp[e] = uc * ic

      @pl.when(g < NG - 1)
      def _():
        s2 = pl.multiple_of((g + 1) * L, L)
        fire(h, uid_v[pl.ds(s2, L)], iid_v[pl.ds(s2, L)])

      acc = ptmp[0] * 0.0
      for l in range(L):
        acc = acc + plsc.load_gather(ptmp, [lanes, jnp.full((L,), l, jnp.int32)])
      if h == 0:
        acc = acc + ub_v[pl.ds(s, L)] + ib_v[pl.ds(s, L)]
      else:
        acc = acc + out_v[pl.ds(s, L)]
      out_v[pl.ds(s, L)] = acc
      return carry

    lax.fori_loop(0, NG, group, 0)

  run_pass(0)
  run_pass(1)

  pltpu.sync_copy(out_v, out_hbm.at[pl.ds(base, BPW)])


@jax.jit
def _run(user_ids, item_ids, ue_t, ie_t, ub, ib):
  mesh = plsc.VectorSubcoreMesh(
      core_axis_name="c", subcore_axis_name="s",
      num_cores=NC, num_subcores=NS)
  f = pl.kernel(
      _sc_body,
      out_type=jax.ShapeDtypeStruct((B,), jnp.float32),
      mesh=mesh,
      scratch_types=[
          pltpu.VMEM((BPW,), jnp.int32),
          pltpu.VMEM((BPW,), jnp.int32),
          pltpu.VMEM((L * HD, W), jnp.float32),
          pltpu.VMEM((L * HD, W), jnp.float32),
          pltpu.VMEM((L, L), jnp.float32),
          pltpu.VMEM((BPW,), jnp.float32),
          pltpu.VMEM((BPW,), jnp.float32),
          pltpu.VMEM((BPW,), jnp.float32),
          pltpu.SemaphoreType.DMA,
          pltpu.SemaphoreType.DMA,
      ],
      compiler_params=pltpu.CompilerParams(
          needs_layout_passes=False, use_tc_tiling_on_sc=True),
  )
  return f(user_ids, item_ids, ue_t, ie_t, ub, ib)


def kernel(user_ids, item_ids, user_emb, item_emb, user_bias, item_bias):
  return _run(user_ids.astype(jnp.int32), item_ids.astype(jnp.int32),
              user_emb.T, item_emb.T,
              user_bias.reshape(-1), item_bias.reshape(-1))


# speedup vs baseline: 15.4363x; 1.0163x over previous
import jax
import jax.numpy as jnp
from jax import lax
from jax.experimental import pallas as pl
from jax.experimental.pallas import tpu as pltpu
from jax.experimental.pallas import tpu_sc as plsc

B = 16384
D = 32
NC, NS = 2, 16
NW = NC * NS
BPW = B // NW
CHUNK = 128
NCHUNK = BPW // CHUNK
L = 16
NG = BPW // L
W = 128
HD = 16
HG = 8


def _sc_body(uid_hbm, iid_hbm, ue_hbm, ie_hbm, ub_hbm, ib_hbm, out_hbm,
             uid_v, iid_v, win_u, win_i, ptmp, ub_v, ib_v, out_v,
             semA, semB, bsem):
  sems = (semA, semB)
  wid = lax.axis_index("s") * NC + lax.axis_index("c")
  base = pl.multiple_of(wid * BPW, BPW)

  pltpu.sync_copy(uid_hbm.at[pl.ds(base, BPW)], uid_v)
  pltpu.sync_copy(iid_hbm.at[pl.ds(base, BPW)], iid_v)

  for j in range(NCHUNK):
    pltpu.async_copy(ub_hbm.at[uid_v.at[pl.ds(j * CHUNK, CHUNK)]],
                     ub_v.at[pl.ds(j * CHUNK, CHUNK)], bsem)
    pltpu.async_copy(ib_hbm.at[iid_v.at[pl.ds(j * CHUNK, CHUNK)]],
                     ib_v.at[pl.ds(j * CHUNK, CHUNK)], bsem)
  pltpu.make_async_copy(ub_hbm.at[pl.ds(0, BPW)], ub_v, bsem).wait()
  pltpu.make_async_copy(ib_hbm.at[pl.ds(0, BPW)], ib_v, bsem).wait()

  lanes = lax.iota(jnp.int32, L)
  lanes8 = lanes & 7

  def fire(h, half, u16, i16):
    ublk = u16 & -128
    iblk = i16 & -128
    dh = h * HD
    for e in range(HG):
      ou = pl.multiple_of(ublk[half * HG + e], W)
      oi = pl.multiple_of(iblk[half * HG + e], W)
      pltpu.async_copy(ue_hbm.at[pl.ds(dh, HD), pl.ds(ou, W)],
                       win_u.at[pl.ds((half * HG + e) * HD, HD)], sems[half])
      pltpu.async_copy(ie_hbm.at[pl.ds(dh, HD), pl.ds(oi, W)],
                       win_i.at[pl.ds((half * HG + e) * HD, HD)], sems[half])

  def extract(half, uoff, ioff):
    for e in range(HG):
      ru = (half * HG + e) * HD + lanes
      uc = plsc.load_gather(
          win_u, [ru, jnp.full((L,), uoff[half * HG + e], jnp.int32)])
      ic = plsc.load_gather(
          win_i, [ru, jnp.full((L,), ioff[half * HG + e], jnp.int32)])
      ptmp[e] = uc * ic
    acc8 = ptmp[0] * 0.0
    for l in range(L):
      acc8 = acc8 + plsc.load_gather(
          ptmp, [lanes8, jnp.full((L,), l, jnp.int32)])
    return acc8

  def drain(half):
    pltpu.make_async_copy(ue_hbm.at[pl.ds(0, HD), pl.ds(0, W)],
                          win_u.at[pl.ds(half * HG * HD, HG * HD)],
                          sems[half]).wait()
    pltpu.make_async_copy(ie_hbm.at[pl.ds(0, HD), pl.ds(0, W)],
                          win_i.at[pl.ds(half * HG * HD, HG * HD)],
                          sems[half]).wait()

  def run_pass(h):
    u0 = uid_v[pl.ds(0, L)]
    i0 = iid_v[pl.ds(0, L)]
    fire(h, 0, u0, i0)
    fire(h, 1, u0, i0)

    def group(g, carry):
      s = pl.multiple_of(g * L, L)
      u16 = uid_v[pl.ds(s, L)]
      i16 = iid_v[pl.ds(s, L)]
      uoff = u16 & 127
      ioff = i16 & 127

      drain(0)
      accA = extract(0, uoff, ioff)

      @pl.when(g < NG - 1)
      def _():
        s2 = pl.multiple_of((g + 1) * L, L)
        fire(h, 0, uid_v[pl.ds(s2, L)], iid_v[pl.ds(s2, L)])

      drain(1)
      accB = extract(1, uoff, ioff)

      @pl.when(g < NG - 1)
      def _():
        s2 = pl.multiple_of((g + 1) * L, L)
        fire(h, 1, uid_v[pl.ds(s2, L)], iid_v[pl.ds(s2, L)])

      acc = jnp.where(lanes < HG, accA, accB)
      if h == 0:
        acc = acc + ub_v[pl.ds(s, L)] + ib_v[pl.ds(s, L)]
      else:
        acc = acc + out_v[pl.ds(s, L)]
      out_v[pl.ds(s, L)] = acc
      return carry

    lax.fori_loop(0, NG, group, 0)

  run_pass(0)
  run_pass(1)

  pltpu.sync_copy(out_v, out_hbm.at[pl.ds(base, BPW)])


@jax.jit
def _run(user_ids, item_ids, ue_t, ie_t, ub, ib):
  mesh = plsc.VectorSubcoreMesh(
      core_axis_name="c", subcore_axis_name="s",
      num_cores=NC, num_subcores=NS)
  f = pl.kernel(
      _sc_body,
      out_type=jax.ShapeDtypeStruct((B,), jnp.float32),
      mesh=mesh,
      scratch_types=[
          pltpu.VMEM((BPW,), jnp.int32),
          pltpu.VMEM((BPW,), jnp.int32),
          pltpu.VMEM((L * HD, W), jnp.float32),
          pltpu.VMEM((L * HD, W), jnp.float32),
          pltpu.VMEM((HG, L), jnp.float32),
          pltpu.VMEM((BPW,), jnp.float32),
          pltpu.VMEM((BPW,), jnp.float32),
          pltpu.VMEM((BPW,), jnp.float32),
          pltpu.SemaphoreType.DMA,
          pltpu.SemaphoreType.DMA,
          pltpu.SemaphoreType.DMA,
      ],
      compiler_params=pltpu.CompilerParams(
          needs_layout_passes=False, use_tc_tiling_on_sc=True),
  )
  return f(user_ids, item_ids, ue_t, ie_t, ub, ib)


def kernel(user_ids, item_ids, user_emb, item_emb, user_bias, item_bias):
  return _run(user_ids.astype(jnp.int32), item_ids.astype(jnp.int32),
              user_emb.T, item_emb.T,
              user_bias.reshape(-1), item_bias.reshape(-1))
